# baseline (speedup 1.0000x reference)
import functools

import jax
import jax.numpy as jnp
from jax import lax
from jax.experimental import pallas as pl
from jax.experimental.pallas import tpu as pltpu
from jax.experimental.pallas import tpu_sc as plsc

MAXLEN = 8192
OUTPUT_DIM = 2048

_NC = 2
_NS = 16
_NW = _NC * _NS
_ROWS_PER_W = MAXLEN // _NW


def _sc_copy(table_hbm, out_hbm):
    wid = lax.axis_index("s") * _NC + lax.axis_index("c")
    base = wid * _ROWS_PER_W
    pltpu.sync_copy(table_hbm.at[pl.ds(base, _ROWS_PER_W)],
                    out_hbm.at[pl.ds(base, _ROWS_PER_W)])


def kernel(inputs, table):
    del inputs
    mesh = plsc.VectorSubcoreMesh(core_axis_name="c", subcore_axis_name="s")
    out = pl.kernel(
        _sc_copy,
        mesh=mesh,
        out_type=jax.ShapeDtypeStruct((MAXLEN, OUTPUT_DIM), table.dtype),
    )(table)
    return out[None]


# speedup vs baseline: 30.2381x; 30.2381x over previous
import jax
import jax.numpy as jnp
from jax import lax
from jax.experimental import pallas as pl
from jax.experimental.pallas import tpu as pltpu
from jax.experimental.pallas import tpu_sc as plsc

MAXLEN = 8192
OUTPUT_DIM = 2048

_NC = 2
_NS = 16
_NW = _NC * _NS
_ROWS_PER_W = MAXLEN // _NW
_CHUNK = 16
_NCHUNKS = _ROWS_PER_W // _CHUNK


def _sc_copy(table_hbm, out_hbm, buf0, buf1, in_s0, in_s1, out_s0, out_s1):
    wid = lax.axis_index("s") * _NC + lax.axis_index("c")
    base = wid * _ROWS_PER_W
    bufs = (buf0, buf1)
    in_sems = (in_s0, in_s1)
    out_sems = (out_s0, out_s1)
    for i in range(_NCHUNKS):
        b = i % 2
        lo = base + i * _CHUNK
        if i >= 2:
            pltpu.make_async_copy(bufs[b], out_hbm.at[pl.ds(lo - 2 * _CHUNK, _CHUNK)],
                                  out_sems[b]).wait()
        cin = pltpu.make_async_copy(table_hbm.at[pl.ds(lo, _CHUNK)], bufs[b],
                                    in_sems[b])
        cin.start()
        cin.wait()
        pltpu.make_async_copy(bufs[b], out_hbm.at[pl.ds(lo, _CHUNK)],
                              out_sems[b]).start()
    for i in range(_NCHUNKS - 2, _NCHUNKS):
        b = i % 2
        lo = base + i * _CHUNK
        pltpu.make_async_copy(bufs[b], out_hbm.at[pl.ds(lo, _CHUNK)],
                              out_sems[b]).wait()


def kernel(inputs, table):
    del inputs
    mesh = plsc.VectorSubcoreMesh(core_axis_name="c", subcore_axis_name="s")
    out = pl.kernel(
        _sc_copy,
        mesh=mesh,
        out_type=jax.ShapeDtypeStruct((MAXLEN, OUTPUT_DIM), table.dtype),
        scratch_types=[
            pltpu.VMEM((_CHUNK, OUTPUT_DIM), jnp.float32),
            pltpu.VMEM((_CHUNK, OUTPUT_DIM), jnp.float32),
            pltpu.SemaphoreType.DMA,
            pltpu.SemaphoreType.DMA,
            pltpu.SemaphoreType.DMA,
            pltpu.SemaphoreType.DMA,
        ],
    )(table)
    return out[None]
